# baseline (speedup 1.0000x reference)
import functools

import jax
import jax.numpy as jnp
from jax import lax
from jax.experimental import pallas as pl
from jax.experimental.pallas import tpu as pltpu
from jax.experimental.pallas import tpu_sc as plsc

NG = 52
NS = NG * NG
NB = 32
NA = 3
NCH = 86
NCHA = NA * NCH
NCLS = 80
NT = 256
NEXC = 4 * NT
NCELL = NB * NA * NS
STRIDE = 416.0 / NG
IMG = 416.0
THR = 0.5
NO_OBJ_W = 10.0


def _prep_body(tg_ref, anc_ref, bun_ref, winx_ref, cidx_ref, sidx_ref):
    tg = tg_ref[...]
    bi = tg[:, 0:1].astype(jnp.int32)
    cx = tg[:, 1:2] * NG
    cy = tg[:, 2:3] * NG
    tw = tg[:, 3:4]
    th = tg[:, 4:5]
    istop = tg[:, 5:6]
    ci = cx.astype(jnp.int32)
    cj = cy.astype(jnp.int32)
    a00, a01 = anc_ref[0, 0], anc_ref[0, 1]
    a10, a11 = anc_ref[1, 0], anc_ref[1, 1]
    a20, a21 = anc_ref[2, 0], anc_ref[2, 1]

    def iou(aw, ah):
        inter = jnp.minimum(aw, tw) * jnp.minimum(ah, th)
        union = tw * th + aw * ah - inter + 1e-16
        return inter / union

    i0, i1, i2 = iou(a00, a01), iou(a10, a11), iou(a20, a21)
    best_a = jnp.where(i0 >= i1, jnp.where(i0 >= i2, 0, 2),
                       jnp.where(i1 >= i2, 1, 2)).astype(jnp.int32)
    spat = cj * NG + ci
    rowid = (bi * NA + best_a) * NS + spat
    tix = lax.broadcasted_iota(jnp.int32, (NT, NT), 0)
    kix = lax.broadcasted_iota(jnp.int32, (NT, NT), 1)
    eq = rowid == rowid.T
    winner = ~jnp.any(eq & (kix > tix), axis=1, keepdims=True)
    topk = (istop > 0.5).T
    any_top = jnp.any(eq & topk, axis=1, keepdims=True)
    any_bot = jnp.any(eq & ~topk, axis=1, keepdims=True)
    tx = cx - jnp.floor(cx)
    ty = cy - jnp.floor(cy)
    aw_b = jnp.where(best_a == 0, a00, jnp.where(best_a == 1, a10, a20))
    ah_b = jnp.where(best_a == 0, a01, jnp.where(best_a == 1, a11, a21))
    twl = jnp.log(tw / aw_b + 1e-16)
    thl = jnp.log(th / ah_b + 1e-16)
    zero = jnp.zeros((NT, 1), jnp.float32)
    bun = jnp.concatenate([
        winner.astype(jnp.float32), any_top.astype(jnp.float32),
        any_bot.astype(jnp.float32), tx, ty, twl, thl,
        (bi % 8).astype(jnp.float32), best_a.astype(jnp.float32),
        zero, zero, zero, zero, zero, zero, zero], axis=1)
    bun_ref[...] = bun
    base = bi * NA * NS + spat
    d0 = base + 0 * NS
    d1 = base + 1 * NS
    d2 = base + 2 * NS
    idall = jnp.concatenate([rowid, d0, d1, d2], axis=0)
    actall = jnp.concatenate(
        [jnp.ones((NT, 1), jnp.bool_), i0 > THR, i1 > THR, i2 > THR], axis=0)
    tix1 = lax.broadcasted_iota(jnp.int32, (NEXC, NEXC), 0)
    kix1 = lax.broadcasted_iota(jnp.int32, (NEXC, NEXC), 1)
    eq1 = (idall == idall.T) & actall & actall.T & (kix1 < tix1)
    winx = actall & ~jnp.any(eq1, axis=1, keepdims=True)
    winx_ref[...] = winx.astype(jnp.float32)
    cbase = bi * (NCHA * NS) + 4 * NS + spat
    cobj = cbase + best_a * (NCH * NS)
    c0 = cbase + 0 * (NCH * NS)
    c1 = cbase + 1 * (NCH * NS)
    c2 = cbase + 2 * (NCH * NS)
    cid = jnp.concatenate([cobj, c0, c1, c2], axis=0)
    cidx_ref[...] = jnp.where(actall, cid, 0)
    sidx_ref[...] = jnp.concatenate([spat, bi // 8], axis=1)


def _prep(target, anchors):
    return pl.pallas_call(
        _prep_body,
        in_specs=[
            pl.BlockSpec((NT, NCH), lambda: (0, 0)),
            pl.BlockSpec(memory_space=pltpu.SMEM),
        ],
        out_specs=[
            pl.BlockSpec((NT, 16), lambda: (0, 0)),
            pl.BlockSpec((NEXC, 1), lambda: (0, 0)),
            pl.BlockSpec((NEXC, 1), lambda: (0, 0)),
            pl.BlockSpec((NT, 2), lambda: (0, 0)),
        ],
        out_shape=[
            jax.ShapeDtypeStruct((NT, 16), jnp.float32),
            jax.ShapeDtypeStruct((NEXC, 1), jnp.float32),
            jax.ShapeDtypeStruct((NEXC, 1), jnp.int32),
            jax.ShapeDtypeStruct((NT, 2), jnp.int32),
        ],
    )(target, anchors)


_SC_NSUB = 16
_SC_IDX = NEXC // _SC_NSUB


def _sc_confgather_body(idx_hbm, x_hbm, out_hbm, idx_v, dst_v, sem):
    tid = lax.axis_index("s")
    pltpu.sync_copy(idx_hbm.at[pl.ds(tid * _SC_IDX, _SC_IDX)], idx_v)
    pltpu.async_copy(x_hbm.at[idx_v], dst_v, sem).wait()
    pltpu.sync_copy(dst_v, out_hbm.at[pl.ds(tid * _SC_IDX, _SC_IDX)])


@functools.lru_cache(maxsize=None)
def _sc_confgather_fn():
    return pl.kernel(
        _sc_confgather_body,
        out_type=jax.ShapeDtypeStruct((NEXC,), jnp.float32),
        mesh=plsc.VectorSubcoreMesh(
            core_axis_name="c", subcore_axis_name="s", num_cores=1),
        scratch_types=[
            pltpu.VMEM((_SC_IDX,), jnp.int32),
            pltpu.VMEM((_SC_IDX,), jnp.float32),
            pltpu.SemaphoreType.DMA,
        ],
        compiler_params=pltpu.CompilerParams(use_tc_tiling_on_sc=False),
    )


def _dense_body(x_ref, anc_ref, out_ref, acc_ref):
    b = pl.program_id(0)
    X = x_ref[0]
    s_iota = lax.broadcasted_iota(jnp.int32, (1, NS), 1)
    gx = (s_iota % NG).astype(jnp.float32)
    gy = (s_iota // NG).astype(jnp.float32)
    cidx = lax.broadcasted_iota(jnp.int32, (NCH, 1), 0)
    pieces = []
    nsum = 0.0
    for a in range(NA):
        Xa = X[a * NCH:(a + 1) * NCH, :]
        Sa = 1.0 / (1.0 + jnp.exp(-Xa))
        Ea = jnp.exp(Xa)
        aw = anc_ref[a, 0]
        ah = anc_ref[a, 1]
        anc = jnp.where(cidx == 2, aw, ah)
        ya = jnp.where(cidx == 0, (gx + Sa) * STRIDE,
             jnp.where(cidx == 1, (gy + Sa) * STRIDE,
             jnp.where(cidx < 4, anc * IMG * Ea, Sa)))
        pieces.append(ya)
        p4 = Sa[4:5, :]
        term = -jnp.maximum(jnp.log(1.0 - p4), -100.0)
        nsum += jnp.sum(term)
    out_ref[...] = jnp.concatenate(pieces, axis=1).reshape(1, NCH, NA * NS)

    @pl.when(b == 0)
    def _():
        acc_ref[0, 0] = 0.0

    acc_ref[0, 0] += nsum


def _dense(x3, anchors):
    return pl.pallas_call(
        _dense_body,
        grid=(NB,),
        in_specs=[
            pl.BlockSpec((1, NCHA, NS), lambda b: (b, 0, 0)),
            pl.BlockSpec(memory_space=pltpu.SMEM),
        ],
        out_specs=[
            pl.BlockSpec((1, NCH, NA * NS), lambda b: (b, 0, 0)),
            pl.BlockSpec(memory_space=pltpu.SMEM),
        ],
        out_shape=[
            jax.ShapeDtypeStruct((NB, NCH, NA * NS), jnp.float32),
            jax.ShapeDtypeStruct((1, 1), jnp.float32),
        ],
    )(x3, anchors)


def _gather_body(sidx_ref, xt_ref, g_ref):
    g_ref[...] = xt_ref[...]


def _gather(sidx, xt):
    return pl.pallas_call(
        _gather_body,
        grid_spec=pltpu.PrefetchScalarGridSpec(
            num_scalar_prefetch=1,
            grid=(NT,),
            in_specs=[
                pl.BlockSpec((1, 8, NCHA), lambda t, s: (s[t, 0], s[t, 1], 0)),
            ],
            out_specs=pl.BlockSpec((1, 8, NCHA), lambda t, s: (t, 0, 0)),
        ),
        out_shape=jax.ShapeDtypeStruct((NT, 8, NCHA), jnp.float32),
    )(sidx, xt)


def _final_body(g_ref, bun_ref, tg_ref, conf_ref, winx_ref, acc_ref, out_ref):
    G = g_ref[...]
    bun = bun_ref[...]
    tg = tg_ref[...]
    win = bun[:, 0:1]
    atop = bun[:, 1:2]
    abot = bun[:, 2:3]
    tx = bun[:, 3:4]
    ty = bun[:, 4:5]
    twl = bun[:, 5:6]
    thl = bun[:, 6:7]
    bm8 = bun[:, 7:8]
    ba = bun[:, 8:9]
    istop = tg[:, 5:6]
    r8 = lax.broadcasted_iota(jnp.int32, (NT, 8, 1), 1).astype(jnp.float32)
    Grow = jnp.sum(jnp.where(r8 == bm8[:, :, None], G, 0.0), axis=1)
    Z = jnp.where(ba == 0.0, Grow[:, 0:NCH],
        jnp.where(ba == 1.0, Grow[:, NCH:2 * NCH],
                  Grow[:, 2 * NCH:3 * NCH]))
    P = 1.0 / (1.0 + jnp.exp(-Z))
    p_cx = P[:, 0:1]
    p_cy = P[:, 1:2]
    z_w = Z[:, 2:3]
    z_h = Z[:, 3:4]
    p_conf = P[:, 4:5]
    p_it = P[:, 5:6]
    Pc = P[:, 6:NCH]
    n_obj = jnp.sum(win)
    den = jnp.maximum(n_obj, 1.0)
    loss_box = jnp.sum(win * ((p_cx - tx) ** 2 + (p_cy - ty) ** 2 +
                              (z_w - twl) ** 2 + (z_h - thl) ** 2)) / den
    conf_obj = jnp.sum(win * (-jnp.maximum(jnp.log(p_conf), -100.0))) / den
    lp5 = jnp.maximum(jnp.log(p_it), -100.0)
    l1p5 = jnp.maximum(jnp.log(1.0 - p_it), -100.0)
    loss_label = jnp.sum(win * (-(istop * lp5 + (1.0 - istop) * l1p5))) / den
    T = tg[:, 6:NCH]
    lpP = jnp.maximum(jnp.log(Pc), -100.0)
    l1pP = jnp.maximum(jnp.log(1.0 - Pc), -100.0)
    L = -(T * lpP + (1.0 - T) * l1pP)
    topm = (lax.broadcasted_iota(jnp.int32, (1, NCLS), 1) < 40).astype(
        jnp.float32)
    top_s = jnp.sum(L * topm, axis=1, keepdims=True)
    bot_s = jnp.sum(L * (1.0 - topm), axis=1, keepdims=True)
    n_topc = jnp.sum(win * atop)
    n_botc = jnp.sum(win * abot)
    loss_top = jnp.where(
        n_topc > 0,
        jnp.sum(win * atop * top_s) / jnp.maximum(40.0 * n_topc, 1.0), 0.0)
    loss_bot = jnp.where(
        n_botc > 0,
        jnp.sum(win * abot * bot_s) / jnp.maximum(40.0 * n_botc, 1.0), 0.0)
    zc = conf_ref[...]
    wx = winx_ref[...]
    pzc = 1.0 / (1.0 + jnp.exp(-zc))
    excl_term = -jnp.maximum(jnp.log(1.0 - pzc), -100.0)
    corr = jnp.sum(wx * excl_term)
    nexc = jnp.sum(wx)
    bce_noobj = (acc_ref[0, 0] - corr) / jnp.maximum(NCELL - nexc, 1.0)
    loss_conf = conf_obj + NO_OBJ_W * bce_noobj
    out_ref[0, 0] = (loss_box + loss_conf + loss_label + loss_top + loss_bot)


def _final(g, bun, target, conf, winx, acc):
    return pl.pallas_call(
        _final_body,
        in_specs=[
            pl.BlockSpec((NT, 8, NCHA), lambda: (0, 0, 0)),
            pl.BlockSpec((NT, 16), lambda: (0, 0)),
            pl.BlockSpec((NT, NCH), lambda: (0, 0)),
            pl.BlockSpec((NEXC, 1), lambda: (0, 0)),
            pl.BlockSpec((NEXC, 1), lambda: (0, 0)),
            pl.BlockSpec(memory_space=pltpu.SMEM),
        ],
        out_specs=pl.BlockSpec(memory_space=pltpu.SMEM),
        out_shape=jax.ShapeDtypeStruct((1, 1), jnp.float32),
    )(g, bun, target, conf, winx, acc)


def kernel(x, target, anchors):
    xt = x.transpose(2, 3, 0, 1).reshape(NS, NB, NCHA)
    x3 = x.reshape(NB, NCHA, NS)
    bun, winx, cidx, sidx = _prep(target, anchors)
    conf = _sc_confgather_fn()(cidx.reshape(NEXC), x.reshape(NB * NCHA * NS))
    out4, acc = _dense(x3, anchors)
    g = _gather(sidx, xt)
    loss = _final(g, bun, target, conf.reshape(NEXC, 1), winx, acc)
    output = out4.transpose(0, 2, 1)
    return output, loss.reshape(())


# speedup vs baseline: 1.5019x; 1.5019x over previous
import functools

import jax
import jax.numpy as jnp
from jax import lax
from jax.experimental import pallas as pl
from jax.experimental.pallas import tpu as pltpu
from jax.experimental.pallas import tpu_sc as plsc

NG = 52
NS = NG * NG
NB = 32
NA = 3
NCH = 86
NCHA = NA * NCH
NCLS = 80
NT = 256
NEXC = 4 * NT
NCELL = NB * NA * NS
STRIDE = 416.0 / NG
IMG = 416.0
THR = 0.5
NO_OBJ_W = 10.0


def _prep_body(tg_ref, anc_ref, bun_ref, winx_ref, cidx_ref, sidx_ref):
    tg = tg_ref[...]
    bi = tg[:, 0:1].astype(jnp.int32)
    cx = tg[:, 1:2] * NG
    cy = tg[:, 2:3] * NG
    tw = tg[:, 3:4]
    th = tg[:, 4:5]
    istop = tg[:, 5:6]
    ci = cx.astype(jnp.int32)
    cj = cy.astype(jnp.int32)
    a00, a01 = anc_ref[0, 0], anc_ref[0, 1]
    a10, a11 = anc_ref[1, 0], anc_ref[1, 1]
    a20, a21 = anc_ref[2, 0], anc_ref[2, 1]

    def iou(aw, ah):
        inter = jnp.minimum(aw, tw) * jnp.minimum(ah, th)
        union = tw * th + aw * ah - inter + 1e-16
        return inter / union

    i0, i1, i2 = iou(a00, a01), iou(a10, a11), iou(a20, a21)
    best_a = jnp.where(i0 >= i1, jnp.where(i0 >= i2, 0, 2),
                       jnp.where(i1 >= i2, 1, 2)).astype(jnp.int32)
    spat = cj * NG + ci
    rowid = (bi * NA + best_a) * NS + spat
    tix = lax.broadcasted_iota(jnp.int32, (NT, NT), 0)
    kix = lax.broadcasted_iota(jnp.int32, (NT, NT), 1)
    eq = rowid == rowid.T
    winner = ~jnp.any(eq & (kix > tix), axis=1, keepdims=True)
    topk = (istop > 0.5).T
    any_top = jnp.any(eq & topk, axis=1, keepdims=True)
    any_bot = jnp.any(eq & ~topk, axis=1, keepdims=True)
    tx = cx - jnp.floor(cx)
    ty = cy - jnp.floor(cy)
    aw_b = jnp.where(best_a == 0, a00, jnp.where(best_a == 1, a10, a20))
    ah_b = jnp.where(best_a == 0, a01, jnp.where(best_a == 1, a11, a21))
    twl = jnp.log(tw / aw_b + 1e-16)
    thl = jnp.log(th / ah_b + 1e-16)
    zero = jnp.zeros((NT, 1), jnp.float32)
    bun = jnp.concatenate([
        winner.astype(jnp.float32), any_top.astype(jnp.float32),
        any_bot.astype(jnp.float32), tx, ty, twl, thl,
        (bi % 8).astype(jnp.float32), best_a.astype(jnp.float32),
        zero, zero, zero, zero, zero, zero, zero], axis=1)
    bun_ref[...] = bun
    base = bi * NA * NS + spat
    d0 = base + 0 * NS
    d1 = base + 1 * NS
    d2 = base + 2 * NS
    idall = jnp.concatenate([rowid, d0, d1, d2], axis=0)
    actall = jnp.concatenate(
        [jnp.ones((NT, 1), jnp.bool_), i0 > THR, i1 > THR, i2 > THR], axis=0)
    tix1 = lax.broadcasted_iota(jnp.int32, (NEXC, NEXC), 0)
    kix1 = lax.broadcasted_iota(jnp.int32, (NEXC, NEXC), 1)
    eq1 = (idall == idall.T) & actall & actall.T & (kix1 < tix1)
    winx = actall & ~jnp.any(eq1, axis=1, keepdims=True)
    winx_ref[...] = winx.astype(jnp.float32)
    cidx_ref[...] = jnp.where(actall, idall, 0)
    sidx_ref[...] = jnp.concatenate([spat, bi // 8], axis=1)


def _prep(target, anchors):
    return pl.pallas_call(
        _prep_body,
        in_specs=[
            pl.BlockSpec((NT, NCH), lambda: (0, 0)),
            pl.BlockSpec(memory_space=pltpu.SMEM),
        ],
        out_specs=[
            pl.BlockSpec((NT, 16), lambda: (0, 0)),
            pl.BlockSpec((NEXC, 1), lambda: (0, 0)),
            pl.BlockSpec((NEXC, 1), lambda: (0, 0)),
            pl.BlockSpec((NT, 2), lambda: (0, 0)),
        ],
        out_shape=[
            jax.ShapeDtypeStruct((NT, 16), jnp.float32),
            jax.ShapeDtypeStruct((NEXC, 1), jnp.float32),
            jax.ShapeDtypeStruct((NEXC, 1), jnp.int32),
            jax.ShapeDtypeStruct((NT, 2), jnp.int32),
        ],
    )(target, anchors)


_SC_NSUB = 16
_SC_IDX = NEXC // _SC_NSUB


def _sc_confgather_body(idx_hbm, x_hbm, out_hbm, idx_v, dst_v, sem):
    tid = lax.axis_index("s")
    pltpu.sync_copy(idx_hbm.at[pl.ds(tid * _SC_IDX, _SC_IDX)], idx_v)
    pltpu.async_copy(x_hbm.at[idx_v], dst_v, sem).wait()
    pltpu.sync_copy(dst_v, out_hbm.at[pl.ds(tid * _SC_IDX, _SC_IDX)])


@functools.lru_cache(maxsize=None)
def _sc_confgather_fn():
    return pl.kernel(
        _sc_confgather_body,
        out_type=jax.ShapeDtypeStruct((NEXC,), jnp.float32),
        mesh=plsc.VectorSubcoreMesh(
            core_axis_name="c", subcore_axis_name="s", num_cores=1),
        scratch_types=[
            pltpu.VMEM((_SC_IDX,), jnp.int32),
            pltpu.VMEM((_SC_IDX,), jnp.float32),
            pltpu.SemaphoreType.DMA,
        ],
        compiler_params=pltpu.CompilerParams(use_tc_tiling_on_sc=False),
    )


def _dense_body(x_ref, anc_ref, out_ref, acc_ref):
    b = pl.program_id(0)
    X = x_ref[0]
    s_iota = lax.broadcasted_iota(jnp.int32, (1, NS), 1)
    gx = (s_iota % NG).astype(jnp.float32)
    gy = (s_iota // NG).astype(jnp.float32)
    cidx = lax.broadcasted_iota(jnp.int32, (NCH, 1), 0)
    pieces = []
    nsum = 0.0
    for a in range(NA):
        Xa = X[a * NCH:(a + 1) * NCH, :]
        Sa = 1.0 / (1.0 + jnp.exp(-Xa))
        Ea = jnp.exp(Xa)
        aw = anc_ref[a, 0]
        ah = anc_ref[a, 1]
        anc = jnp.where(cidx == 2, aw, ah)
        ya = jnp.where(cidx == 0, (gx + Sa) * STRIDE,
             jnp.where(cidx == 1, (gy + Sa) * STRIDE,
             jnp.where(cidx < 4, anc * IMG * Ea, Sa)))
        pieces.append(ya)
        p4 = Sa[4:5, :]
        term = -jnp.maximum(jnp.log(1.0 - p4), -100.0)
        nsum += jnp.sum(term)
    out_ref[...] = jnp.concatenate(pieces, axis=1).reshape(1, NCH, NA * NS)

    @pl.when(b == 0)
    def _():
        acc_ref[0, 0] = 0.0

    acc_ref[0, 0] += nsum


def _dense(x3, anchors):
    return pl.pallas_call(
        _dense_body,
        grid=(NB,),
        in_specs=[
            pl.BlockSpec((1, NCHA, NS), lambda b: (b, 0, 0)),
            pl.BlockSpec(memory_space=pltpu.SMEM),
        ],
        out_specs=[
            pl.BlockSpec((1, NCH, NA * NS), lambda b: (b, 0, 0)),
            pl.BlockSpec(memory_space=pltpu.SMEM),
        ],
        out_shape=[
            jax.ShapeDtypeStruct((NB, NCH, NA * NS), jnp.float32),
            jax.ShapeDtypeStruct((1, 1), jnp.float32),
        ],
    )(x3, anchors)


def _gather_body(sidx_ref, xt_ref, g_ref):
    g_ref[...] = xt_ref[...]


def _gather(sidx, xt):
    return pl.pallas_call(
        _gather_body,
        grid_spec=pltpu.PrefetchScalarGridSpec(
            num_scalar_prefetch=1,
            grid=(NT,),
            in_specs=[
                pl.BlockSpec((1, 8, NCHA), lambda t, s: (s[t, 0], s[t, 1], 0)),
            ],
            out_specs=pl.BlockSpec((1, 8, NCHA), lambda t, s: (t, 0, 0)),
        ),
        out_shape=jax.ShapeDtypeStruct((NT, 8, NCHA), jnp.float32),
    )(sidx, xt)


def _final_body(g_ref, bun_ref, tg_ref, conf_ref, winx_ref, acc_ref, out_ref):
    G = g_ref[...]
    bun = bun_ref[...]
    tg = tg_ref[...]
    win = bun[:, 0:1]
    atop = bun[:, 1:2]
    abot = bun[:, 2:3]
    tx = bun[:, 3:4]
    ty = bun[:, 4:5]
    twl = bun[:, 5:6]
    thl = bun[:, 6:7]
    bm8 = bun[:, 7:8]
    ba = bun[:, 8:9]
    istop = tg[:, 5:6]
    r8 = lax.broadcasted_iota(jnp.int32, (NT, 8, 1), 1).astype(jnp.float32)
    Grow = jnp.sum(jnp.where(r8 == bm8[:, :, None], G, 0.0), axis=1)
    Z = jnp.where(ba == 0.0, Grow[:, 0:NCH],
        jnp.where(ba == 1.0, Grow[:, NCH:2 * NCH],
                  Grow[:, 2 * NCH:3 * NCH]))
    P = 1.0 / (1.0 + jnp.exp(-Z))
    p_cx = P[:, 0:1]
    p_cy = P[:, 1:2]
    z_w = Z[:, 2:3]
    z_h = Z[:, 3:4]
    p_conf = P[:, 4:5]
    p_it = P[:, 5:6]
    Pc = P[:, 6:NCH]
    n_obj = jnp.sum(win)
    den = jnp.maximum(n_obj, 1.0)
    loss_box = jnp.sum(win * ((p_cx - tx) ** 2 + (p_cy - ty) ** 2 +
                              (z_w - twl) ** 2 + (z_h - thl) ** 2)) / den
    conf_obj = jnp.sum(win * (-jnp.maximum(jnp.log(p_conf), -100.0))) / den
    lp5 = jnp.maximum(jnp.log(p_it), -100.0)
    l1p5 = jnp.maximum(jnp.log(1.0 - p_it), -100.0)
    loss_label = jnp.sum(win * (-(istop * lp5 + (1.0 - istop) * l1p5))) / den
    T = tg[:, 6:NCH]
    lpP = jnp.maximum(jnp.log(Pc), -100.0)
    l1pP = jnp.maximum(jnp.log(1.0 - Pc), -100.0)
    L = -(T * lpP + (1.0 - T) * l1pP)
    topm = (lax.broadcasted_iota(jnp.int32, (1, NCLS), 1) < 40).astype(
        jnp.float32)
    top_s = jnp.sum(L * topm, axis=1, keepdims=True)
    bot_s = jnp.sum(L * (1.0 - topm), axis=1, keepdims=True)
    n_topc = jnp.sum(win * atop)
    n_botc = jnp.sum(win * abot)
    loss_top = jnp.where(
        n_topc > 0,
        jnp.sum(win * atop * top_s) / jnp.maximum(40.0 * n_topc, 1.0), 0.0)
    loss_bot = jnp.where(
        n_botc > 0,
        jnp.sum(win * abot * bot_s) / jnp.maximum(40.0 * n_botc, 1.0), 0.0)
    zc = conf_ref[...]
    wx = winx_ref[...]
    pzc = 1.0 / (1.0 + jnp.exp(-zc))
    excl_term = -jnp.maximum(jnp.log(1.0 - pzc), -100.0)
    corr = jnp.sum(wx * excl_term)
    nexc = jnp.sum(wx)
    bce_noobj = (acc_ref[0, 0] - corr) / jnp.maximum(NCELL - nexc, 1.0)
    loss_conf = conf_obj + NO_OBJ_W * bce_noobj
    out_ref[0, 0] = (loss_box + loss_conf + loss_label + loss_top + loss_bot)


def _final(g, bun, target, conf, winx, acc):
    return pl.pallas_call(
        _final_body,
        in_specs=[
            pl.BlockSpec((NT, 8, NCHA), lambda: (0, 0, 0)),
            pl.BlockSpec((NT, 16), lambda: (0, 0)),
            pl.BlockSpec((NT, NCH), lambda: (0, 0)),
            pl.BlockSpec((NEXC, 1), lambda: (0, 0)),
            pl.BlockSpec((NEXC, 1), lambda: (0, 0)),
            pl.BlockSpec(memory_space=pltpu.SMEM),
        ],
        out_specs=pl.BlockSpec(memory_space=pltpu.SMEM),
        out_shape=jax.ShapeDtypeStruct((1, 1), jnp.float32),
    )(g, bun, target, conf, winx, acc)


def kernel(x, target, anchors):
    xt = x.transpose(2, 3, 0, 1).reshape(NS, NB, NCHA)
    x3 = x.reshape(NB, NCHA, NS)
    bun, winx, cidx, sidx = _prep(target, anchors)
    xconf = x[:, 4::NCH, :, :].reshape(NCELL)
    conf = _sc_confgather_fn()(cidx.reshape(NEXC), xconf)
    out4, acc = _dense(x3, anchors)
    g = _gather(sidx, xt)
    loss = _final(g, bun, target, conf.reshape(NEXC, 1), winx, acc)
    output = out4.transpose(0, 2, 1)
    return output, loss.reshape(())


# speedup vs baseline: 2.6803x; 1.7845x over previous
import functools

import jax
import jax.numpy as jnp
from jax import lax
from jax.experimental import pallas as pl
from jax.experimental.pallas import tpu as pltpu
from jax.experimental.pallas import tpu_sc as plsc

NG = 52
NS = NG * NG
NB = 32
NA = 3
NCH = 86
NCHA = NA * NCH
NCLS = 80
NT = 256
NEXC = 4 * NT
NCELL = NB * NA * NS
STRIDE = 416.0 / NG
IMG = 416.0
THR = 0.5
NO_OBJ_W = 10.0


def _prep_body(tg_ref, anc_ref, bun_ref, winx_ref, cidx_ref, sidx_ref):
    tg = tg_ref[...]
    bi = tg[:, 0:1].astype(jnp.int32)
    cx = tg[:, 1:2] * NG
    cy = tg[:, 2:3] * NG
    tw = tg[:, 3:4]
    th = tg[:, 4:5]
    istop = tg[:, 5:6]
    ci = cx.astype(jnp.int32)
    cj = cy.astype(jnp.int32)
    a00, a01 = anc_ref[0, 0], anc_ref[0, 1]
    a10, a11 = anc_ref[1, 0], anc_ref[1, 1]
    a20, a21 = anc_ref[2, 0], anc_ref[2, 1]

    def iou(aw, ah):
        inter = jnp.minimum(aw, tw) * jnp.minimum(ah, th)
        union = tw * th + aw * ah - inter + 1e-16
        return inter / union

    i0, i1, i2 = iou(a00, a01), iou(a10, a11), iou(a20, a21)
    best_a = jnp.where(i0 >= i1, jnp.where(i0 >= i2, 0, 2),
                       jnp.where(i1 >= i2, 1, 2)).astype(jnp.int32)
    spat = cj * NG + ci
    rowid = (bi * NA + best_a) * NS + spat
    tix = lax.broadcasted_iota(jnp.int32, (NT, NT), 0)
    kix = lax.broadcasted_iota(jnp.int32, (NT, NT), 1)
    eq = rowid == rowid.T
    winner = ~jnp.any(eq & (kix > tix), axis=1, keepdims=True)
    topk = (istop > 0.5).T
    any_top = jnp.any(eq & topk, axis=1, keepdims=True)
    any_bot = jnp.any(eq & ~topk, axis=1, keepdims=True)
    tx = cx - jnp.floor(cx)
    ty = cy - jnp.floor(cy)
    aw_b = jnp.where(best_a == 0, a00, jnp.where(best_a == 1, a10, a20))
    ah_b = jnp.where(best_a == 0, a01, jnp.where(best_a == 1, a11, a21))
    twl = jnp.log(tw / aw_b + 1e-16)
    thl = jnp.log(th / ah_b + 1e-16)
    zero = jnp.zeros((NT, 1), jnp.float32)
    bun = jnp.concatenate([
        winner.astype(jnp.float32), any_top.astype(jnp.float32),
        any_bot.astype(jnp.float32), tx, ty, twl, thl,
        (bi % 8).astype(jnp.float32), best_a.astype(jnp.float32),
        zero, zero, zero, zero, zero, zero, zero], axis=1)
    bun_ref[...] = bun
    base = bi * NA * NS + spat
    d0 = base + 0 * NS
    d1 = base + 1 * NS
    d2 = base + 2 * NS
    idall = jnp.concatenate([rowid, d0, d1, d2], axis=0)
    actall = jnp.concatenate(
        [jnp.ones((NT, 1), jnp.bool_), i0 > THR, i1 > THR, i2 > THR], axis=0)
    tix1 = lax.broadcasted_iota(jnp.int32, (NEXC, NEXC), 0)
    kix1 = lax.broadcasted_iota(jnp.int32, (NEXC, NEXC), 1)
    eq1 = (idall == idall.T) & actall & actall.T & (kix1 < tix1)
    winx = actall & ~jnp.any(eq1, axis=1, keepdims=True)
    winx_ref[...] = winx.astype(jnp.float32)
    cidx_ref[...] = jnp.where(actall, idall, 0)
    sidx_ref[...] = jnp.concatenate([spat, bi // 8], axis=1)


def _prep(target, anchors):
    return pl.pallas_call(
        _prep_body,
        in_specs=[
            pl.BlockSpec((NT, NCH), lambda: (0, 0)),
            pl.BlockSpec(memory_space=pltpu.SMEM),
        ],
        out_specs=[
            pl.BlockSpec((NT, 16), lambda: (0, 0)),
            pl.BlockSpec((NEXC, 1), lambda: (0, 0)),
            pl.BlockSpec((NEXC, 1), lambda: (0, 0)),
            pl.BlockSpec((NT, 2), lambda: (0, 0)),
        ],
        out_shape=[
            jax.ShapeDtypeStruct((NT, 16), jnp.float32),
            jax.ShapeDtypeStruct((NEXC, 1), jnp.float32),
            jax.ShapeDtypeStruct((NEXC, 1), jnp.int32),
            jax.ShapeDtypeStruct((NT, 2), jnp.int32),
        ],
    )(target, anchors)


_SC_NSUB = 16
_SC_IDX = NEXC // _SC_NSUB


def _sc_confgather_body(idx_hbm, x_hbm, out_hbm, idx_v, dst_v, sem):
    tid = lax.axis_index("s")
    pltpu.sync_copy(idx_hbm.at[pl.ds(tid * _SC_IDX, _SC_IDX)], idx_v)
    pltpu.async_copy(x_hbm.at[idx_v], dst_v, sem).wait()
    pltpu.sync_copy(dst_v, out_hbm.at[pl.ds(tid * _SC_IDX, _SC_IDX)])


@functools.lru_cache(maxsize=None)
def _sc_confgather_fn():
    return pl.kernel(
        _sc_confgather_body,
        out_type=jax.ShapeDtypeStruct((NEXC,), jnp.float32),
        mesh=plsc.VectorSubcoreMesh(
            core_axis_name="c", subcore_axis_name="s", num_cores=1),
        scratch_types=[
            pltpu.VMEM((_SC_IDX,), jnp.int32),
            pltpu.VMEM((_SC_IDX,), jnp.float32),
            pltpu.SemaphoreType.DMA,
        ],
        compiler_params=pltpu.CompilerParams(use_tc_tiling_on_sc=False),
    )


def _dense_body(x_ref, anc_ref, out_ref, cf_ref, acc_ref):
    b = pl.program_id(0)
    X = x_ref[0]
    s_iota = lax.broadcasted_iota(jnp.int32, (1, NS), 1)
    gx = (s_iota % NG).astype(jnp.float32)
    gy = (s_iota // NG).astype(jnp.float32)
    cidx = lax.broadcasted_iota(jnp.int32, (NCH, 1), 0)
    pieces = []
    nsum = 0.0
    for a in range(NA):
        Xa = X[a * NCH:(a + 1) * NCH, :]
        Sa = 1.0 / (1.0 + jnp.exp(-Xa))
        Ea = jnp.exp(Xa)
        aw = anc_ref[a, 0]
        ah = anc_ref[a, 1]
        anc = jnp.where(cidx == 2, aw, ah)
        ya = jnp.where(cidx == 0, (gx + Sa) * STRIDE,
             jnp.where(cidx == 1, (gy + Sa) * STRIDE,
             jnp.where(cidx < 4, anc * IMG * Ea, Sa)))
        pieces.append(ya)
        p4 = Sa[4:5, :]
        term = -jnp.maximum(jnp.log(1.0 - p4), -100.0)
        nsum += jnp.sum(term)
    out_ref[...] = jnp.concatenate(pieces, axis=1).reshape(1, NCH, NA * NS)
    cf_ref[...] = jnp.concatenate(
        [X[a * NCH + 4:a * NCH + 5, :] for a in range(NA)],
        axis=1).reshape(1, 1, NA * NS)

    @pl.when(b == 0)
    def _():
        acc_ref[0, 0] = 0.0

    acc_ref[0, 0] += nsum


def _dense(x3, anchors):
    return pl.pallas_call(
        _dense_body,
        grid=(NB,),
        in_specs=[
            pl.BlockSpec((1, NCHA, NS), lambda b: (b, 0, 0)),
            pl.BlockSpec(memory_space=pltpu.SMEM),
        ],
        out_specs=[
            pl.BlockSpec((1, NCH, NA * NS), lambda b: (b, 0, 0)),
            pl.BlockSpec((1, 1, NA * NS), lambda b: (b, 0, 0)),
            pl.BlockSpec(memory_space=pltpu.SMEM),
        ],
        out_shape=[
            jax.ShapeDtypeStruct((NB, NCH, NA * NS), jnp.float32),
            jax.ShapeDtypeStruct((NB, 1, NA * NS), jnp.float32),
            jax.ShapeDtypeStruct((1, 1), jnp.float32),
        ],
    )(x3, anchors)


def _gather_body(sidx_ref, xt_ref, g_ref):
    g_ref[...] = xt_ref[...]


def _gather(sidx, xt):
    return pl.pallas_call(
        _gather_body,
        grid_spec=pltpu.PrefetchScalarGridSpec(
            num_scalar_prefetch=1,
            grid=(NT,),
            in_specs=[
                pl.BlockSpec((1, 8, NCHA), lambda t, s: (s[t, 0], s[t, 1], 0)),
            ],
            out_specs=pl.BlockSpec((1, 8, NCHA), lambda t, s: (t, 0, 0)),
        ),
        out_shape=jax.ShapeDtypeStruct((NT, 8, NCHA), jnp.float32),
    )(sidx, xt)


def _final_body(g_ref, bun_ref, tg_ref, conf_ref, winx_ref, acc_ref, out_ref):
    G = g_ref[...]
    bun = bun_ref[...]
    tg = tg_ref[...]
    win = bun[:, 0:1]
    atop = bun[:, 1:2]
    abot = bun[:, 2:3]
    tx = bun[:, 3:4]
    ty = bun[:, 4:5]
    twl = bun[:, 5:6]
    thl = bun[:, 6:7]
    bm8 = bun[:, 7:8]
    ba = bun[:, 8:9]
    istop = tg[:, 5:6]
    r8 = lax.broadcasted_iota(jnp.int32, (NT, 8, 1), 1).astype(jnp.float32)
    Grow = jnp.sum(jnp.where(r8 == bm8[:, :, None], G, 0.0), axis=1)
    Z = jnp.where(ba == 0.0, Grow[:, 0:NCH],
        jnp.where(ba == 1.0, Grow[:, NCH:2 * NCH],
                  Grow[:, 2 * NCH:3 * NCH]))
    P = 1.0 / (1.0 + jnp.exp(-Z))
    p_cx = P[:, 0:1]
    p_cy = P[:, 1:2]
    z_w = Z[:, 2:3]
    z_h = Z[:, 3:4]
    p_conf = P[:, 4:5]
    p_it = P[:, 5:6]
    Pc = P[:, 6:NCH]
    n_obj = jnp.sum(win)
    den = jnp.maximum(n_obj, 1.0)
    loss_box = jnp.sum(win * ((p_cx - tx) ** 2 + (p_cy - ty) ** 2 +
                              (z_w - twl) ** 2 + (z_h - thl) ** 2)) / den
    conf_obj = jnp.sum(win * (-jnp.maximum(jnp.log(p_conf), -100.0))) / den
    lp5 = jnp.maximum(jnp.log(p_it), -100.0)
    l1p5 = jnp.maximum(jnp.log(1.0 - p_it), -100.0)
    loss_label = jnp.sum(win * (-(istop * lp5 + (1.0 - istop) * l1p5))) / den
    T = tg[:, 6:NCH]
    lpP = jnp.maximum(jnp.log(Pc), -100.0)
    l1pP = jnp.maximum(jnp.log(1.0 - Pc), -100.0)
    L = -(T * lpP + (1.0 - T) * l1pP)
    topm = (lax.broadcasted_iota(jnp.int32, (1, NCLS), 1) < 40).astype(
        jnp.float32)
    top_s = jnp.sum(L * topm, axis=1, keepdims=True)
    bot_s = jnp.sum(L * (1.0 - topm), axis=1, keepdims=True)
    n_topc = jnp.sum(win * atop)
    n_botc = jnp.sum(win * abot)
    loss_top = jnp.where(
        n_topc > 0,
        jnp.sum(win * atop * top_s) / jnp.maximum(40.0 * n_topc, 1.0), 0.0)
    loss_bot = jnp.where(
        n_botc > 0,
        jnp.sum(win * abot * bot_s) / jnp.maximum(40.0 * n_botc, 1.0), 0.0)
    zc = conf_ref[...]
    wx = winx_ref[...]
    pzc = 1.0 / (1.0 + jnp.exp(-zc))
    excl_term = -jnp.maximum(jnp.log(1.0 - pzc), -100.0)
    corr = jnp.sum(wx * excl_term)
    nexc = jnp.sum(wx)
    bce_noobj = (acc_ref[0, 0] - corr) / jnp.maximum(NCELL - nexc, 1.0)
    loss_conf = conf_obj + NO_OBJ_W * bce_noobj
    out_ref[0, 0] = (loss_box + loss_conf + loss_label + loss_top + loss_bot)


def _final(g, bun, target, conf, winx, acc):
    return pl.pallas_call(
        _final_body,
        in_specs=[
            pl.BlockSpec((NT, 8, NCHA), lambda: (0, 0, 0)),
            pl.BlockSpec((NT, 16), lambda: (0, 0)),
            pl.BlockSpec((NT, NCH), lambda: (0, 0)),
            pl.BlockSpec((NEXC, 1), lambda: (0, 0)),
            pl.BlockSpec((NEXC, 1), lambda: (0, 0)),
            pl.BlockSpec(memory_space=pltpu.SMEM),
        ],
        out_specs=pl.BlockSpec(memory_space=pltpu.SMEM),
        out_shape=jax.ShapeDtypeStruct((1, 1), jnp.float32),
    )(g, bun, target, conf, winx, acc)


def kernel(x, target, anchors):
    xt = x.transpose(2, 3, 0, 1).reshape(NS, NB, NCHA)
    x3 = x.reshape(NB, NCHA, NS)
    bun, winx, cidx, sidx = _prep(target, anchors)
    out4, cf, acc = _dense(x3, anchors)
    conf = _sc_confgather_fn()(cidx.reshape(NEXC), cf.reshape(NCELL))
    g = _gather(sidx, xt)
    loss = _final(g, bun, target, conf.reshape(NEXC, 1), winx, acc)
    output = out4.transpose(0, 2, 1)
    return output, loss.reshape(())


# speedup vs baseline: 3.4423x; 1.2843x over previous
import functools

import jax
import jax.numpy as jnp
from jax import lax
from jax.experimental import pallas as pl
from jax.experimental.pallas import tpu as pltpu
from jax.experimental.pallas import tpu_sc as plsc

NG = 52
NS = NG * NG
NB = 32
NA = 3
NCH = 86
NCHA = NA * NCH
NCLS = 80
NT = 256
NEXC = 4 * NT
NCELL = NB * NA * NS
STRIDE = 416.0 / NG
IMG = 416.0
THR = 0.5
NO_OBJ_W = 10.0


def _prep_body(tg_ref, anc_ref, bun_ref, winx_ref, cidx_ref, sidx_ref):
    tg = tg_ref[...]
    bi = tg[:, 0:1].astype(jnp.int32)
    cx = tg[:, 1:2] * NG
    cy = tg[:, 2:3] * NG
    tw = tg[:, 3:4]
    th = tg[:, 4:5]
    istop = tg[:, 5:6]
    ci = cx.astype(jnp.int32)
    cj = cy.astype(jnp.int32)
    a00, a01 = anc_ref[0, 0], anc_ref[0, 1]
    a10, a11 = anc_ref[1, 0], anc_ref[1, 1]
    a20, a21 = anc_ref[2, 0], anc_ref[2, 1]

    def iou(aw, ah):
        inter = jnp.minimum(aw, tw) * jnp.minimum(ah, th)
        union = tw * th + aw * ah - inter + 1e-16
        return inter / union

    i0, i1, i2 = iou(a00, a01), iou(a10, a11), iou(a20, a21)
    best_a = jnp.where(i0 >= i1, jnp.where(i0 >= i2, 0, 2),
                       jnp.where(i1 >= i2, 1, 2)).astype(jnp.int32)
    spat = cj * NG + ci
    rowid = (bi * NA + best_a) * NS + spat
    tix = lax.broadcasted_iota(jnp.int32, (NT, NT), 0)
    kix = lax.broadcasted_iota(jnp.int32, (NT, NT), 1)
    eq = rowid == rowid.T
    winner = ~jnp.any(eq & (kix > tix), axis=1, keepdims=True)
    topk = (istop > 0.5).T
    any_top = jnp.any(eq & topk, axis=1, keepdims=True)
    any_bot = jnp.any(eq & ~topk, axis=1, keepdims=True)
    tx = cx - jnp.floor(cx)
    ty = cy - jnp.floor(cy)
    aw_b = jnp.where(best_a == 0, a00, jnp.where(best_a == 1, a10, a20))
    ah_b = jnp.where(best_a == 0, a01, jnp.where(best_a == 1, a11, a21))
    twl = jnp.log(tw / aw_b + 1e-16)
    thl = jnp.log(th / ah_b + 1e-16)
    zero = jnp.zeros((NT, 1), jnp.float32)
    bun = jnp.concatenate([
        winner.astype(jnp.float32), any_top.astype(jnp.float32),
        any_bot.astype(jnp.float32), tx, ty, twl, thl,
        (bi % 8).astype(jnp.float32), best_a.astype(jnp.float32),
        zero, zero, zero, zero, zero, zero, zero], axis=1)
    bun_ref[...] = bun
    base = bi * NA * NS + spat
    d0 = base + 0 * NS
    d1 = base + 1 * NS
    d2 = base + 2 * NS
    idall = jnp.concatenate([rowid, d0, d1, d2], axis=0)
    actall = jnp.concatenate(
        [jnp.ones((NT, 1), jnp.bool_), i0 > THR, i1 > THR, i2 > THR], axis=0)
    tix1 = lax.broadcasted_iota(jnp.int32, (NEXC, NEXC), 0)
    kix1 = lax.broadcasted_iota(jnp.int32, (NEXC, NEXC), 1)
    eq1 = (idall == idall.T) & actall & actall.T & (kix1 < tix1)
    winx = actall & ~jnp.any(eq1, axis=1, keepdims=True)
    winx_ref[...] = winx.astype(jnp.float32)
    cidx_ref[...] = jnp.where(actall, idall, 0)
    sidx_ref[...] = jnp.concatenate([spat, bi // 8], axis=1)


def _prep(target, anchors):
    return pl.pallas_call(
        _prep_body,
        in_specs=[
            pl.BlockSpec((NT, NCH), lambda: (0, 0)),
            pl.BlockSpec(memory_space=pltpu.SMEM),
        ],
        out_specs=[
            pl.BlockSpec((NT, 16), lambda: (0, 0)),
            pl.BlockSpec((NEXC, 1), lambda: (0, 0)),
            pl.BlockSpec((NEXC, 1), lambda: (0, 0)),
            pl.BlockSpec((NT, 2), lambda: (0, 0)),
        ],
        out_shape=[
            jax.ShapeDtypeStruct((NT, 16), jnp.float32),
            jax.ShapeDtypeStruct((NEXC, 1), jnp.float32),
            jax.ShapeDtypeStruct((NEXC, 1), jnp.int32),
            jax.ShapeDtypeStruct((NT, 2), jnp.int32),
        ],
    )(target, anchors)


_SC_NSUB = 16
_SC_IDX = NEXC // _SC_NSUB


def _sc_confgather_body(idx_hbm, x_hbm, out_hbm, idx_v, dst_v, sem):
    tid = lax.axis_index("s")
    pltpu.sync_copy(idx_hbm.at[pl.ds(tid * _SC_IDX, _SC_IDX)], idx_v)
    pltpu.async_copy(x_hbm.at[idx_v], dst_v, sem).wait()
    pltpu.sync_copy(dst_v, out_hbm.at[pl.ds(tid * _SC_IDX, _SC_IDX)])


@functools.lru_cache(maxsize=None)
def _sc_confgather_fn():
    return pl.kernel(
        _sc_confgather_body,
        out_type=jax.ShapeDtypeStruct((NEXC,), jnp.float32),
        mesh=plsc.VectorSubcoreMesh(
            core_axis_name="c", subcore_axis_name="s", num_cores=1),
        scratch_types=[
            pltpu.VMEM((_SC_IDX,), jnp.int32),
            pltpu.VMEM((_SC_IDX,), jnp.float32),
            pltpu.SemaphoreType.DMA,
        ],
        compiler_params=pltpu.CompilerParams(use_tc_tiling_on_sc=False),
    )


def _dense_body(x_ref, anc_ref, out_ref, cf_ref, acc_ref):
    b = pl.program_id(0)
    X = x_ref[0]
    s_iota = lax.broadcasted_iota(jnp.int32, (1, NS), 1)
    gx = (s_iota % NG).astype(jnp.float32)
    gy = (s_iota // NG).astype(jnp.float32)
    cidx = lax.broadcasted_iota(jnp.int32, (NCH, 1), 0)
    pieces = []
    nsum = 0.0
    for a in range(NA):
        Xa = X[a * NCH:(a + 1) * NCH, :]
        Sa = 1.0 / (1.0 + jnp.exp(-Xa))
        Ea = jnp.exp(Xa)
        aw = anc_ref[a, 0]
        ah = anc_ref[a, 1]
        anc = jnp.where(cidx == 2, aw, ah)
        ya = jnp.where(cidx == 0, (gx + Sa) * STRIDE,
             jnp.where(cidx == 1, (gy + Sa) * STRIDE,
             jnp.where(cidx < 4, anc * IMG * Ea, Sa)))
        pieces.append(ya)
        p4 = Sa[4:5, :]
        term = -jnp.maximum(jnp.log(1.0 - p4), -100.0)
        nsum += jnp.sum(term)
    out_ref[...] = jnp.concatenate(pieces, axis=1).reshape(1, NCH, NA * NS)
    cf_ref[...] = jnp.concatenate(
        [X[a * NCH + 4:a * NCH + 5, :] for a in range(NA)],
        axis=1).reshape(1, 1, NA * NS)

    @pl.when(b == 0)
    def _():
        acc_ref[0, 0] = 0.0

    acc_ref[0, 0] += nsum


def _dense(x3, anchors):
    return pl.pallas_call(
        _dense_body,
        grid=(NB,),
        in_specs=[
            pl.BlockSpec((1, NCHA, NS), lambda b: (b, 0, 0)),
            pl.BlockSpec(memory_space=pltpu.SMEM),
        ],
        out_specs=[
            pl.BlockSpec((1, NCH, NA * NS), lambda b: (b, 0, 0)),
            pl.BlockSpec((1, 1, NA * NS), lambda b: (b, 0, 0)),
            pl.BlockSpec(memory_space=pltpu.SMEM),
        ],
        out_shape=[
            jax.ShapeDtypeStruct((NB, NCH, NA * NS), jnp.float32),
            jax.ShapeDtypeStruct((NB, 1, NA * NS), jnp.float32),
            jax.ShapeDtypeStruct((1, 1), jnp.float32),
        ],
    )(x3, anchors)


_GB = 8


def _gather_body(sidx_ref, *refs):
    g_ref = refs[_GB]
    for j in range(_GB):
        g_ref[j] = refs[j][0]


def _gather(sidx, xt):
    def mk(j):
        return pl.BlockSpec(
            (1, 8, NCHA), lambda t, s, j=j: (s[t * _GB + j, 0],
                                             s[t * _GB + j, 1], 0))

    return pl.pallas_call(
        _gather_body,
        grid_spec=pltpu.PrefetchScalarGridSpec(
            num_scalar_prefetch=1,
            grid=(NT // _GB,),
            in_specs=[mk(j) for j in range(_GB)],
            out_specs=pl.BlockSpec((_GB, 8, NCHA), lambda t, s: (t, 0, 0)),
        ),
        out_shape=jax.ShapeDtypeStruct((NT, 8, NCHA), jnp.float32),
    )(sidx, *([xt] * _GB))


def _final_body(g_ref, bun_ref, tg_ref, conf_ref, winx_ref, acc_ref, out_ref):
    G = g_ref[...]
    bun = bun_ref[...]
    tg = tg_ref[...]
    win = bun[:, 0:1]
    atop = bun[:, 1:2]
    abot = bun[:, 2:3]
    tx = bun[:, 3:4]
    ty = bun[:, 4:5]
    twl = bun[:, 5:6]
    thl = bun[:, 6:7]
    bm8 = bun[:, 7:8]
    ba = bun[:, 8:9]
    istop = tg[:, 5:6]
    r8 = lax.broadcasted_iota(jnp.int32, (NT, 8, 1), 1).astype(jnp.float32)
    Grow = jnp.sum(jnp.where(r8 == bm8[:, :, None], G, 0.0), axis=1)
    Z = jnp.where(ba == 0.0, Grow[:, 0:NCH],
        jnp.where(ba == 1.0, Grow[:, NCH:2 * NCH],
                  Grow[:, 2 * NCH:3 * NCH]))
    P = 1.0 / (1.0 + jnp.exp(-Z))
    p_cx = P[:, 0:1]
    p_cy = P[:, 1:2]
    z_w = Z[:, 2:3]
    z_h = Z[:, 3:4]
    p_conf = P[:, 4:5]
    p_it = P[:, 5:6]
    Pc = P[:, 6:NCH]
    n_obj = jnp.sum(win)
    den = jnp.maximum(n_obj, 1.0)
    loss_box = jnp.sum(win * ((p_cx - tx) ** 2 + (p_cy - ty) ** 2 +
                              (z_w - twl) ** 2 + (z_h - thl) ** 2)) / den
    conf_obj = jnp.sum(win * (-jnp.maximum(jnp.log(p_conf), -100.0))) / den
    lp5 = jnp.maximum(jnp.log(p_it), -100.0)
    l1p5 = jnp.maximum(jnp.log(1.0 - p_it), -100.0)
    loss_label = jnp.sum(win * (-(istop * lp5 + (1.0 - istop) * l1p5))) / den
    T = tg[:, 6:NCH]
    lpP = jnp.maximum(jnp.log(Pc), -100.0)
    l1pP = jnp.maximum(jnp.log(1.0 - Pc), -100.0)
    L = -(T * lpP + (1.0 - T) * l1pP)
    topm = (lax.broadcasted_iota(jnp.int32, (1, NCLS), 1) < 40).astype(
        jnp.float32)
    top_s = jnp.sum(L * topm, axis=1, keepdims=True)
    bot_s = jnp.sum(L * (1.0 - topm), axis=1, keepdims=True)
    n_topc = jnp.sum(win * atop)
    n_botc = jnp.sum(win * abot)
    loss_top = jnp.where(
        n_topc > 0,
        jnp.sum(win * atop * top_s) / jnp.maximum(40.0 * n_topc, 1.0), 0.0)
    loss_bot = jnp.where(
        n_botc > 0,
        jnp.sum(win * abot * bot_s) / jnp.maximum(40.0 * n_botc, 1.0), 0.0)
    zc = conf_ref[...]
    wx = winx_ref[...]
    pzc = 1.0 / (1.0 + jnp.exp(-zc))
    excl_term = -jnp.maximum(jnp.log(1.0 - pzc), -100.0)
    corr = jnp.sum(wx * excl_term)
    nexc = jnp.sum(wx)
    bce_noobj = (acc_ref[0, 0] - corr) / jnp.maximum(NCELL - nexc, 1.0)
    loss_conf = conf_obj + NO_OBJ_W * bce_noobj
    out_ref[0, 0] = (loss_box + loss_conf + loss_label + loss_top + loss_bot)


def _final(g, bun, target, conf, winx, acc):
    return pl.pallas_call(
        _final_body,
        in_specs=[
            pl.BlockSpec((NT, 8, NCHA), lambda: (0, 0, 0)),
            pl.BlockSpec((NT, 16), lambda: (0, 0)),
            pl.BlockSpec((NT, NCH), lambda: (0, 0)),
            pl.BlockSpec((NEXC, 1), lambda: (0, 0)),
            pl.BlockSpec((NEXC, 1), lambda: (0, 0)),
            pl.BlockSpec(memory_space=pltpu.SMEM),
        ],
        out_specs=pl.BlockSpec(memory_space=pltpu.SMEM),
        out_shape=jax.ShapeDtypeStruct((1, 1), jnp.float32),
    )(g, bun, target, conf, winx, acc)


def kernel(x, target, anchors):
    xt = x.transpose(2, 3, 0, 1).reshape(NS, NB, NCHA)
    x3 = x.reshape(NB, NCHA, NS)
    bun, winx, cidx, sidx = _prep(target, anchors)
    out4, cf, acc = _dense(x3, anchors)
    conf = _sc_confgather_fn()(cidx.reshape(NEXC), cf.reshape(NCELL))
    g = _gather(sidx, xt)
    loss = _final(g, bun, target, conf.reshape(NEXC, 1), winx, acc)
    output = out4.transpose(0, 2, 1)
    return output, loss.reshape(())
